# baseline (speedup 1.0000x reference)
import functools
import math

import jax
import jax.numpy as jnp
import numpy as np
from jax import lax
from jax.experimental import pallas as pl
from jax.experimental.pallas import tpu as pltpu
from jax.experimental.pallas import tpu_sc as plsc

N = 10000
E = 320000
NUM_CAT = 32
EMB = 128
PE = 64
H = 128
L = 4

NP = 10240
R = 1000
GRID = N // R

NTILES = 32
CHUNK = 64
NBUF = 2
NCH0 = 272
NCH1 = 48
EP = 16 * (NCH0 + NCH1) * CHUNK
ZR = NP // 16


def _mm(a, b):
    return jnp.dot(a, b, preferred_element_type=jnp.float32)


def _gelu(x):
    return 0.5 * x * (1.0 + lax.erf(x * (1.0 / math.sqrt(2.0))))


def _pe_table():
    pos = np.arange(PE, dtype=np.float32)[:, None]
    div = np.exp(np.arange(0, PE, 2, dtype=np.float32) * (-math.log(10000.0) / PE))
    tab = np.concatenate([np.sin(pos * div), np.cos(pos * div)], axis=-1)
    return jnp.asarray(tab, dtype=jnp.float32)


def _k0_body(x_ref, al_ref, emb_ref, pet_ref, w1e_ref, w1p_ref, b1_ref,
             w2_ref, b2_ref, wm_ref, bm_ref, wt_ref, bt_ref, ws_ref, bs_ref,
             h_ref, m1_ref, m2_ref, s_ref):
    x = x_ref[...]
    al = al_ref[...]
    ohe = (x == lax.broadcasted_iota(jnp.int32, (R, NUM_CAT), 1)).astype(jnp.float32)
    ohp = (al == lax.broadcasted_iota(jnp.int32, (R, PE), 1)).astype(jnp.float32)
    e = _mm(ohe, emb_ref[...])
    p = _mm(ohp, pet_ref[...])
    g = _gelu(_mm(e, w1e_ref[...]) + _mm(p, w1p_ref[...]) + b1_ref[...])
    h = _mm(g, w2_ref[...]) + b2_ref[...]
    h_ref[...] = h
    m1_ref[...] = _mm(h, wm_ref[...]) + bm_ref[...]
    m2_ref[...] = _mm(h, wt_ref[...]) + bt_ref[...]
    s_ref[...] = _mm(h, ws_ref[...]) + bs_ref[...]


def _kmid_body(agg_ref, sp_ref, wm_ref, bm_ref, wt_ref, bt_ref, ws_ref, bs_ref,
               h_ref, m1_ref, m2_ref, s_ref):
    h = _gelu(agg_ref[0] + agg_ref[1] + sp_ref[...])
    h_ref[...] = h
    m1_ref[...] = _mm(h, wm_ref[...]) + bm_ref[...]
    m2_ref[...] = _mm(h, wt_ref[...]) + bt_ref[...]
    s_ref[...] = _mm(h, ws_ref[...]) + bs_ref[...]


def _kfin_body(h0_ref, h1_ref, h2_ref, h3_ref, agg_ref, sp_ref,
               wo1_ref, bo1_ref, wo2_ref, bo2_ref, out_ref):
    h4 = _gelu(agg_ref[0] + agg_ref[1] + sp_ref[...])
    w = wo1_ref[...]
    t = (_mm(h0_ref[...], w[0:H]) + _mm(h1_ref[...], w[H:2 * H])
         + _mm(h2_ref[...], w[2 * H:3 * H]) + _mm(h3_ref[...], w[3 * H:4 * H])
         + _mm(h4, w[4 * H:5 * H]) + bo1_ref[...])
    out_ref[...] = _mm(_gelu(t), wo2_ref[...]) + bo2_ref[...]


def _node_spec(cols=H):
    return pl.BlockSpec((R, cols), lambda i: (i, 0))


def _full_spec(shape):
    nd = len(shape)
    return pl.BlockSpec(shape, lambda i, _nd=nd: (0,) * _nd)


def _tc_call(body, in_specs, out_shapes, out_specs):
    return pl.pallas_call(
        body,
        grid=(GRID,),
        in_specs=in_specs,
        out_specs=out_specs,
        out_shape=out_shapes,
    )


_sc_mesh = plsc.VectorSubcoreMesh(core_axis_name="c", subcore_axis_name="s")


@functools.partial(
    pl.kernel,
    out_type=jax.ShapeDtypeStruct((2, NP, H), jnp.float32),
    mesh=_sc_mesh,
    scratch_types=[
        [pltpu.VMEM((2 * CHUNK,), jnp.int32) for _ in range(2)],
        [pltpu.VMEM((2, CHUNK), jnp.int32) for _ in range(NBUF)],
        [pltpu.VMEM((CHUNK, H), jnp.float32) for _ in range(NBUF)],
        [pltpu.VMEM((CHUNK, H), jnp.float32) for _ in range(NBUF)],
        pltpu.VMEM_SHARED((NP, H), jnp.float32),
        [pltpu.SemaphoreType.DMA for _ in range(NBUF)],
        [pltpu.SemaphoreType.DMA for _ in range(NBUF)],
        [pltpu.SemaphoreType.DMA for _ in range(NBUF)],
        [pltpu.SemaphoreType.DMA for _ in range(NBUF)],
        [pltpu.SemaphoreType.DMA for _ in range(2)],
    ],
)
def _edge_agg(m1_hbm, m2_hbm, edges_hbm, zeros_hbm, out_hbm,
              pbuf, ubuf, rows1, rows2, acc, g1, g2, s1, s2, pi):
    cid = lax.axis_index("c")
    sid = lax.axis_index("s")
    nch = jnp.where(cid == 0, NCH0, NCH1)
    cbase = jnp.where(cid == 0, sid * NCH0, 16 * NCH0 + sid * NCH1)
    pltpu.sync_copy(zeros_hbm, acc.at[pl.ds(sid * ZR, ZR)])

    def prefetch(k, pb):
        pltpu.async_copy(edges_hbm.at[pl.ds((cbase + k + 2) * CHUNK, 2 * CHUNK)],
                         pbuf[pb], pi[pb])

    def pwait(pb):
        pltpu.make_async_copy(edges_hbm.at[pl.ds(0, 2 * CHUNK)],
                              pbuf[pb], pi[pb]).wait()

    def unpack(b, pb):
        for t in range(CHUNK // 16):
            w = pbuf[pb][pl.ds(b * CHUNK + t * 16, 16)]
            ubuf[b][0, pl.ds(t * 16, 16)] = w & 0xFFFF
            ubuf[b][1, pl.ds(t * 16, 16)] = lax.shift_right_logical(w, 16)

    def gather(b):
        pltpu.async_copy(m1_hbm.at[ubuf[b].at[0]], rows1[b], g1[b])
        pltpu.async_copy(m2_hbm.at[ubuf[b].at[1]], rows2[b], g2[b])

    def gather_wait(b):
        pltpu.make_async_copy(m1_hbm.at[ubuf[b].at[0]], rows1[b], g1[b]).wait()
        pltpu.make_async_copy(m2_hbm.at[ubuf[b].at[1]], rows2[b], g2[b]).wait()

    def scatter(b):
        pltpu.async_copy(rows1[b], acc.at[ubuf[b].at[1]], s1[b], add=True)
        pltpu.async_copy(rows2[b], acc.at[ubuf[b].at[0]], s2[b], add=True)

    def scatter_wait(b):
        pltpu.make_async_copy(rows1[b], acc.at[ubuf[b].at[1]], s1[b]).wait()
        pltpu.make_async_copy(rows2[b], acc.at[ubuf[b].at[0]], s2[b]).wait()

    def visit(k, b, pb_unpack, pb_wait=None, pb_pre=None,
              first=False, last=False):
        gather_wait(b)
        scatter(b)
        if not first:
            scatter_wait(1 - b)
        if not last:
            if pb_wait is not None:
                pwait(pb_wait)
            unpack(1 - b, pb_unpack)
            gather(1 - b)
        if pb_pre is not None:
            prefetch(k, pb_pre)

    pltpu.sync_copy(edges_hbm.at[pl.ds(cbase * CHUNK, 2 * CHUNK)], pbuf[0])
    unpack(0, 0)
    gather(0)
    plsc.subcore_barrier()
    visit(0, 0, pb_unpack=0, pb_pre=1, first=True)
    visit(1, 1, pb_unpack=1, pb_wait=1)

    def outer(j, carry):
        k = 2 + 4 * j
        visit(k, 0, pb_unpack=1, pb_pre=0)
        visit(k + 1, 1, pb_unpack=0, pb_wait=0)
        visit(k + 2, 0, pb_unpack=0, pb_pre=1)
        visit(k + 3, 1, pb_unpack=1, pb_wait=1)
        return carry

    lax.fori_loop(0, (nch - 4) // 4, outer, 0)
    visit(0, 0, pb_unpack=1)
    visit(0, 1, pb_unpack=0, last=True)
    scatter_wait(1)

    plsc.subcore_barrier()
    pltpu.sync_copy(acc.at[pl.ds(sid * ZR, ZR)],
                    out_hbm.at[cid, pl.ds(sid * ZR, ZR)])


def kernel(x_n, edge_index, abs_level, rel_level, emb, Wp1, bp1, Wp2, bp2,
           Wm, bm, Wt, bt, Ws, bs, Wo1, bo1, Wo2, bo2):
    del rel_level
    f32 = jnp.float32
    xc = x_n.reshape(N, 1).astype(jnp.int32)
    al = abs_level.reshape(N, 1).astype(jnp.int32)
    pet = _pe_table()

    src = edge_index[0].astype(jnp.int32)
    dst = edge_index[1].astype(jnp.int32)
    pad = jnp.full((EP - E,), N, dtype=jnp.int32)
    srcp = jnp.concatenate([src, pad])
    dstp = jnp.concatenate([dst, pad])
    edges = srcp | (dstp << 16)
    zeros = jnp.zeros((ZR, H), dtype=f32)

    b1 = bp1.reshape(1, H)
    b2 = bp2.reshape(1, H)
    w1e = Wp1[:EMB]
    w1p = Wp1[EMB:]

    node_i32 = pl.BlockSpec((R, 1), lambda i: (i, 0))
    nspec = _node_spec()
    agg_spec = pl.BlockSpec((2, R, H), lambda i: (0, i, 0))
    mshape = jax.ShapeDtypeStruct((NP, H), f32)
    nshape = jax.ShapeDtypeStruct((N, H), f32)

    h0, m1, m2, s = _tc_call(
        _k0_body,
        in_specs=[node_i32, node_i32, _full_spec((NUM_CAT, H)), _full_spec((PE, PE)),
                  _full_spec((EMB, H)), _full_spec((PE, H)), _full_spec((1, H)),
                  _full_spec((H, H)), _full_spec((1, H)),
                  _full_spec((H, H)), _full_spec((1, H)),
                  _full_spec((H, H)), _full_spec((1, H)),
                  _full_spec((H, H)), _full_spec((1, H))],
        out_shapes=(nshape, mshape, mshape, nshape),
        out_specs=(nspec, nspec, nspec, nspec),
    )(xc, al, emb, pet, w1e, w1p, b1, Wp2, b2,
      Wm[0], bm[0].reshape(1, H), Wt[0], bt[0].reshape(1, H),
      Ws[0], bs[0].reshape(1, H))

    hs = [h0]
    for i in range(L - 1):
        agg = _edge_agg(m1, m2, edges, zeros)
        h, m1, m2, s = _tc_call(
            _kmid_body,
            in_specs=[agg_spec, nspec,
                      _full_spec((H, H)), _full_spec((1, H)),
                      _full_spec((H, H)), _full_spec((1, H)),
                      _full_spec((H, H)), _full_spec((1, H))],
            out_shapes=(nshape, mshape, mshape, nshape),
            out_specs=(nspec, nspec, nspec, nspec),
        )(agg, s, Wm[i + 1], bm[i + 1].reshape(1, H),
          Wt[i + 1], bt[i + 1].reshape(1, H),
          Ws[i + 1], bs[i + 1].reshape(1, H))
        hs.append(h)

    agg = _edge_agg(m1, m2, edges, zeros)
    out = _tc_call(
        _kfin_body,
        in_specs=[nspec, nspec, nspec, nspec, agg_spec, nspec,
                  _full_spec(((L + 1) * H, H)), _full_spec((1, H)),
                  _full_spec((H, H)), _full_spec((1, H))],
        out_shapes=nshape,
        out_specs=nspec,
    )(hs[0], hs[1], hs[2], hs[3], agg, s,
      Wo1, bo1.reshape(1, H), Wo2, bo2.reshape(1, H))
    return out


# speedup vs baseline: 1.0130x; 1.0130x over previous
import functools
import math

import jax
import jax.numpy as jnp
import numpy as np
from jax import lax
from jax.experimental import pallas as pl
from jax.experimental.pallas import tpu as pltpu
from jax.experimental.pallas import tpu_sc as plsc

N = 10000
E = 320000
NUM_CAT = 32
EMB = 128
PE = 64
H = 128
L = 4

NP = 10240
R = 1000
GRID = N // R

NTILES = 32
CHUNK = 80
NBUF = 2
NCH0 = 204
NCH1 = 52
EP = 16 * (NCH0 + NCH1) * CHUNK
ZR = NP // 16


def _mm(a, b):
    return jnp.dot(a, b, preferred_element_type=jnp.float32)


def _gelu(x):
    return 0.5 * x * (1.0 + lax.erf(x * (1.0 / math.sqrt(2.0))))


def _pe_table():
    pos = np.arange(PE, dtype=np.float32)[:, None]
    div = np.exp(np.arange(0, PE, 2, dtype=np.float32) * (-math.log(10000.0) / PE))
    tab = np.concatenate([np.sin(pos * div), np.cos(pos * div)], axis=-1)
    return jnp.asarray(tab, dtype=jnp.float32)


def _k0_body(x_ref, al_ref, emb_ref, pet_ref, w1e_ref, w1p_ref, b1_ref,
             w2_ref, b2_ref, wm_ref, bm_ref, wt_ref, bt_ref, ws_ref, bs_ref,
             h_ref, m1_ref, m2_ref, s_ref):
    x = x_ref[...]
    al = al_ref[...]
    ohe = (x == lax.broadcasted_iota(jnp.int32, (R, NUM_CAT), 1)).astype(jnp.float32)
    ohp = (al == lax.broadcasted_iota(jnp.int32, (R, PE), 1)).astype(jnp.float32)
    e = _mm(ohe, emb_ref[...])
    p = _mm(ohp, pet_ref[...])
    g = _gelu(_mm(e, w1e_ref[...]) + _mm(p, w1p_ref[...]) + b1_ref[...])
    h = _mm(g, w2_ref[...]) + b2_ref[...]
    h_ref[...] = h
    m1_ref[...] = _mm(h, wm_ref[...]) + bm_ref[...]
    m2_ref[...] = _mm(h, wt_ref[...]) + bt_ref[...]
    s_ref[...] = _mm(h, ws_ref[...]) + bs_ref[...]


def _kmid_body(agg_ref, sp_ref, wm_ref, bm_ref, wt_ref, bt_ref, ws_ref, bs_ref,
               h_ref, m1_ref, m2_ref, s_ref):
    h = _gelu(agg_ref[0] + agg_ref[1] + sp_ref[...])
    h_ref[...] = h
    m1_ref[...] = _mm(h, wm_ref[...]) + bm_ref[...]
    m2_ref[...] = _mm(h, wt_ref[...]) + bt_ref[...]
    s_ref[...] = _mm(h, ws_ref[...]) + bs_ref[...]


def _kfin_body(h0_ref, h1_ref, h2_ref, h3_ref, agg_ref, sp_ref,
               wo1_ref, bo1_ref, wo2_ref, bo2_ref, out_ref):
    h4 = _gelu(agg_ref[0] + agg_ref[1] + sp_ref[...])
    w = wo1_ref[...]
    t = (_mm(h0_ref[...], w[0:H]) + _mm(h1_ref[...], w[H:2 * H])
         + _mm(h2_ref[...], w[2 * H:3 * H]) + _mm(h3_ref[...], w[3 * H:4 * H])
         + _mm(h4, w[4 * H:5 * H]) + bo1_ref[...])
    out_ref[...] = _mm(_gelu(t), wo2_ref[...]) + bo2_ref[...]


def _node_spec(cols=H):
    return pl.BlockSpec((R, cols), lambda i: (i, 0))


def _full_spec(shape):
    nd = len(shape)
    return pl.BlockSpec(shape, lambda i, _nd=nd: (0,) * _nd)


def _tc_call(body, in_specs, out_shapes, out_specs):
    return pl.pallas_call(
        body,
        grid=(GRID,),
        in_specs=in_specs,
        out_specs=out_specs,
        out_shape=out_shapes,
    )


_sc_mesh = plsc.VectorSubcoreMesh(core_axis_name="c", subcore_axis_name="s")


@functools.partial(
    pl.kernel,
    out_type=jax.ShapeDtypeStruct((2, NP, H), jnp.float32),
    mesh=_sc_mesh,
    scratch_types=[
        [pltpu.VMEM((2 * CHUNK,), jnp.int32) for _ in range(2)],
        [pltpu.VMEM((2, CHUNK), jnp.int32) for _ in range(NBUF)],
        [pltpu.VMEM((CHUNK, H), jnp.float32) for _ in range(NBUF)],
        [pltpu.VMEM((CHUNK, H), jnp.float32) for _ in range(NBUF)],
        pltpu.VMEM_SHARED((NP, H), jnp.float32),
        [pltpu.SemaphoreType.DMA for _ in range(NBUF)],
        [pltpu.SemaphoreType.DMA for _ in range(NBUF)],
        [pltpu.SemaphoreType.DMA for _ in range(NBUF)],
        [pltpu.SemaphoreType.DMA for _ in range(NBUF)],
        [pltpu.SemaphoreType.DMA for _ in range(2)],
    ],
)
def _edge_agg(m1_hbm, m2_hbm, edges_hbm, zeros_hbm, out_hbm,
              pbuf, ubuf, rows1, rows2, acc, g1, g2, s1, s2, pi):
    cid = lax.axis_index("c")
    sid = lax.axis_index("s")
    nch = jnp.where(cid == 0, NCH0, NCH1)
    cbase = jnp.where(cid == 0, sid * NCH0, 16 * NCH0 + sid * NCH1)
    pltpu.sync_copy(zeros_hbm, acc.at[pl.ds(sid * ZR, ZR)])

    def prefetch(k, pb):
        pltpu.async_copy(edges_hbm.at[pl.ds((cbase + k + 2) * CHUNK, 2 * CHUNK)],
                         pbuf[pb], pi[pb])

    def pwait(pb):
        pltpu.make_async_copy(edges_hbm.at[pl.ds(0, 2 * CHUNK)],
                              pbuf[pb], pi[pb]).wait()

    def unpack(b, pb):
        for t in range(CHUNK // 16):
            w = pbuf[pb][pl.ds(b * CHUNK + t * 16, 16)]
            ubuf[b][0, pl.ds(t * 16, 16)] = w & 0xFFFF
            ubuf[b][1, pl.ds(t * 16, 16)] = lax.shift_right_logical(w, 16)

    def gather(b):
        pltpu.async_copy(m1_hbm.at[ubuf[b].at[0]], rows1[b], g1[b])
        pltpu.async_copy(m2_hbm.at[ubuf[b].at[1]], rows2[b], g2[b])

    def gather_wait(b):
        pltpu.make_async_copy(m1_hbm.at[ubuf[b].at[0]], rows1[b], g1[b]).wait()
        pltpu.make_async_copy(m2_hbm.at[ubuf[b].at[1]], rows2[b], g2[b]).wait()

    def scatter(b):
        pltpu.async_copy(rows1[b], acc.at[ubuf[b].at[1]], s1[b], add=True)
        pltpu.async_copy(rows2[b], acc.at[ubuf[b].at[0]], s2[b], add=True)

    def scatter_wait(b):
        pltpu.make_async_copy(rows1[b], acc.at[ubuf[b].at[1]], s1[b]).wait()
        pltpu.make_async_copy(rows2[b], acc.at[ubuf[b].at[0]], s2[b]).wait()

    def visit(k, b, pb_unpack, pb_wait=None, pb_pre=None,
              first=False, last=False):
        gather_wait(b)
        scatter(b)
        if not first:
            scatter_wait(1 - b)
        if not last:
            if pb_wait is not None:
                pwait(pb_wait)
            unpack(1 - b, pb_unpack)
            gather(1 - b)
        if pb_pre is not None:
            prefetch(k, pb_pre)

    pltpu.sync_copy(edges_hbm.at[pl.ds(cbase * CHUNK, 2 * CHUNK)], pbuf[0])
    unpack(0, 0)
    gather(0)
    plsc.subcore_barrier()
    visit(0, 0, pb_unpack=0, pb_pre=1, first=True)
    visit(1, 1, pb_unpack=1, pb_wait=1)

    def outer(j, carry):
        k = 2 + 4 * j
        visit(k, 0, pb_unpack=1, pb_pre=0)
        visit(k + 1, 1, pb_unpack=0, pb_wait=0)
        visit(k + 2, 0, pb_unpack=0, pb_pre=1)
        visit(k + 3, 1, pb_unpack=1, pb_wait=1)
        return carry

    lax.fori_loop(0, (nch - 4) // 4, outer, 0)
    visit(0, 0, pb_unpack=1)
    visit(0, 1, pb_unpack=0, last=True)
    scatter_wait(1)

    plsc.subcore_barrier()
    pltpu.sync_copy(acc.at[pl.ds(sid * ZR, ZR)],
                    out_hbm.at[cid, pl.ds(sid * ZR, ZR)])


def kernel(x_n, edge_index, abs_level, rel_level, emb, Wp1, bp1, Wp2, bp2,
           Wm, bm, Wt, bt, Ws, bs, Wo1, bo1, Wo2, bo2):
    del rel_level
    f32 = jnp.float32
    xc = x_n.reshape(N, 1).astype(jnp.int32)
    al = abs_level.reshape(N, 1).astype(jnp.int32)
    pet = _pe_table()

    src = edge_index[0].astype(jnp.int32)
    dst = edge_index[1].astype(jnp.int32)
    pad = jnp.full((EP - E,), N, dtype=jnp.int32)
    srcp = jnp.concatenate([src, pad])
    dstp = jnp.concatenate([dst, pad])
    edges = srcp | (dstp << 16)
    zeros = jnp.zeros((ZR, H), dtype=f32)

    b1 = bp1.reshape(1, H)
    b2 = bp2.reshape(1, H)
    w1e = Wp1[:EMB]
    w1p = Wp1[EMB:]

    node_i32 = pl.BlockSpec((R, 1), lambda i: (i, 0))
    nspec = _node_spec()
    agg_spec = pl.BlockSpec((2, R, H), lambda i: (0, i, 0))
    mshape = jax.ShapeDtypeStruct((NP, H), f32)
    nshape = jax.ShapeDtypeStruct((N, H), f32)

    h0, m1, m2, s = _tc_call(
        _k0_body,
        in_specs=[node_i32, node_i32, _full_spec((NUM_CAT, H)), _full_spec((PE, PE)),
                  _full_spec((EMB, H)), _full_spec((PE, H)), _full_spec((1, H)),
                  _full_spec((H, H)), _full_spec((1, H)),
                  _full_spec((H, H)), _full_spec((1, H)),
                  _full_spec((H, H)), _full_spec((1, H)),
                  _full_spec((H, H)), _full_spec((1, H))],
        out_shapes=(nshape, mshape, mshape, nshape),
        out_specs=(nspec, nspec, nspec, nspec),
    )(xc, al, emb, pet, w1e, w1p, b1, Wp2, b2,
      Wm[0], bm[0].reshape(1, H), Wt[0], bt[0].reshape(1, H),
      Ws[0], bs[0].reshape(1, H))

    hs = [h0]
    for i in range(L - 1):
        agg = _edge_agg(m1, m2, edges, zeros)
        h, m1, m2, s = _tc_call(
            _kmid_body,
            in_specs=[agg_spec, nspec,
                      _full_spec((H, H)), _full_spec((1, H)),
                      _full_spec((H, H)), _full_spec((1, H)),
                      _full_spec((H, H)), _full_spec((1, H))],
            out_shapes=(nshape, mshape, mshape, nshape),
            out_specs=(nspec, nspec, nspec, nspec),
        )(agg, s, Wm[i + 1], bm[i + 1].reshape(1, H),
          Wt[i + 1], bt[i + 1].reshape(1, H),
          Ws[i + 1], bs[i + 1].reshape(1, H))
        hs.append(h)

    agg = _edge_agg(m1, m2, edges, zeros)
    out = _tc_call(
        _kfin_body,
        in_specs=[nspec, nspec, nspec, nspec, agg_spec, nspec,
                  _full_spec(((L + 1) * H, H)), _full_spec((1, H)),
                  _full_spec((H, H)), _full_spec((1, H))],
        out_shapes=nshape,
        out_specs=nspec,
    )(hs[0], hs[1], hs[2], hs[3], agg, s,
      Wo1, bo1.reshape(1, H), Wo2, bo2.reshape(1, H))
    return out


# speedup vs baseline: 1.0289x; 1.0157x over previous
import functools
import math

import jax
import jax.numpy as jnp
import numpy as np
from jax import lax
from jax.experimental import pallas as pl
from jax.experimental.pallas import tpu as pltpu
from jax.experimental.pallas import tpu_sc as plsc

N = 10000
E = 320000
NUM_CAT = 32
EMB = 128
PE = 64
H = 128
L = 4

NP = 10240
R = 1000
GRID = N // R

NTILES = 32
CHUNK = 64
NBUF = 2
NCH0 = 256
NCH1 = 64
EP = 16 * (NCH0 + NCH1) * CHUNK
ZR = NP // 16


def _mm(a, b):
    return jnp.dot(a, b, preferred_element_type=jnp.float32)


def _gelu(x):
    return 0.5 * x * (1.0 + lax.erf(x * (1.0 / math.sqrt(2.0))))


def _pe_table():
    pos = np.arange(PE, dtype=np.float32)[:, None]
    div = np.exp(np.arange(0, PE, 2, dtype=np.float32) * (-math.log(10000.0) / PE))
    tab = np.concatenate([np.sin(pos * div), np.cos(pos * div)], axis=-1)
    return jnp.asarray(tab, dtype=jnp.float32)


def _k0_body(x_ref, al_ref, emb_ref, pet_ref, w1e_ref, w1p_ref, b1_ref,
             w2_ref, b2_ref, wm_ref, bm_ref, wt_ref, bt_ref, ws_ref, bs_ref,
             h_ref, m1_ref, m2_ref, s_ref):
    x = x_ref[...]
    al = al_ref[...]
    ohe = (x == lax.broadcasted_iota(jnp.int32, (R, NUM_CAT), 1)).astype(jnp.float32)
    ohp = (al == lax.broadcasted_iota(jnp.int32, (R, PE), 1)).astype(jnp.float32)
    e = _mm(ohe, emb_ref[...])
    p = _mm(ohp, pet_ref[...])
    g = _gelu(_mm(e, w1e_ref[...]) + _mm(p, w1p_ref[...]) + b1_ref[...])
    h = _mm(g, w2_ref[...]) + b2_ref[...]
    h_ref[...] = h
    m1_ref[...] = _mm(h, wm_ref[...]) + bm_ref[...]
    m2_ref[...] = _mm(h, wt_ref[...]) + bt_ref[...]
    s_ref[...] = _mm(h, ws_ref[...]) + bs_ref[...]


def _kmid_body(agg_ref, sp_ref, wm_ref, bm_ref, wt_ref, bt_ref, ws_ref, bs_ref,
               h_ref, m1_ref, m2_ref, s_ref):
    h = _gelu(agg_ref[0] + agg_ref[1] + sp_ref[...])
    h_ref[...] = h
    m1_ref[...] = _mm(h, wm_ref[...]) + bm_ref[...]
    m2_ref[...] = _mm(h, wt_ref[...]) + bt_ref[...]
    s_ref[...] = _mm(h, ws_ref[...]) + bs_ref[...]


def _kfin_body(h0_ref, h1_ref, h2_ref, h3_ref, agg_ref, sp_ref,
               wo1_ref, bo1_ref, wo2_ref, bo2_ref, out_ref):
    h4 = _gelu(agg_ref[0] + agg_ref[1] + sp_ref[...])
    w = wo1_ref[...]
    t = (_mm(h0_ref[...], w[0:H]) + _mm(h1_ref[...], w[H:2 * H])
         + _mm(h2_ref[...], w[2 * H:3 * H]) + _mm(h3_ref[...], w[3 * H:4 * H])
         + _mm(h4, w[4 * H:5 * H]) + bo1_ref[...])
    out_ref[...] = _mm(_gelu(t), wo2_ref[...]) + bo2_ref[...]


def _node_spec(cols=H):
    return pl.BlockSpec((R, cols), lambda i: (i, 0))


def _full_spec(shape):
    nd = len(shape)
    return pl.BlockSpec(shape, lambda i, _nd=nd: (0,) * _nd)


def _tc_call(body, in_specs, out_shapes, out_specs):
    return pl.pallas_call(
        body,
        grid=(GRID,),
        in_specs=in_specs,
        out_specs=out_specs,
        out_shape=out_shapes,
    )


_sc_mesh = plsc.VectorSubcoreMesh(core_axis_name="c", subcore_axis_name="s")


@functools.partial(
    pl.kernel,
    out_type=jax.ShapeDtypeStruct((2, NP, H), jnp.float32),
    mesh=_sc_mesh,
    scratch_types=[
        [pltpu.VMEM((2 * CHUNK,), jnp.int32) for _ in range(2)],
        [pltpu.VMEM((2, CHUNK), jnp.int32) for _ in range(NBUF)],
        [pltpu.VMEM((CHUNK, H), jnp.float32) for _ in range(NBUF)],
        [pltpu.VMEM((CHUNK, H), jnp.float32) for _ in range(NBUF)],
        pltpu.VMEM_SHARED((NP, H), jnp.float32),
        [pltpu.SemaphoreType.DMA for _ in range(NBUF)],
        [pltpu.SemaphoreType.DMA for _ in range(NBUF)],
        [pltpu.SemaphoreType.DMA for _ in range(NBUF)],
        [pltpu.SemaphoreType.DMA for _ in range(NBUF)],
        [pltpu.SemaphoreType.DMA for _ in range(2)],
    ],
)
def _edge_agg(m1_hbm, m2_hbm, edges_hbm, zeros_hbm, out_hbm,
              pbuf, ubuf, rows1, rows2, acc, g1, g2, s1, s2, pi):
    cid = lax.axis_index("c")
    sid = lax.axis_index("s")
    nch = jnp.where(cid == 0, NCH0, NCH1)
    cbase = jnp.where(cid == 0, sid * NCH0, 16 * NCH0 + sid * NCH1)
    pltpu.sync_copy(zeros_hbm, acc.at[pl.ds(sid * ZR, ZR)])

    def prefetch(k, pb):
        pltpu.async_copy(edges_hbm.at[pl.ds((cbase + k + 2) * CHUNK, 2 * CHUNK)],
                         pbuf[pb], pi[pb])

    def pwait(pb):
        pltpu.make_async_copy(edges_hbm.at[pl.ds(0, 2 * CHUNK)],
                              pbuf[pb], pi[pb]).wait()

    def unpack(b, pb):
        for t in range(CHUNK // 16):
            w = pbuf[pb][pl.ds(b * CHUNK + t * 16, 16)]
            ubuf[b][0, pl.ds(t * 16, 16)] = w & 0xFFFF
            ubuf[b][1, pl.ds(t * 16, 16)] = lax.shift_right_logical(w, 16)

    def gather(b):
        pltpu.async_copy(m1_hbm.at[ubuf[b].at[0]], rows1[b], g1[b])
        pltpu.async_copy(m2_hbm.at[ubuf[b].at[1]], rows2[b], g2[b])

    def gather_wait(b):
        pltpu.make_async_copy(m1_hbm.at[ubuf[b].at[0]], rows1[b], g1[b]).wait()
        pltpu.make_async_copy(m2_hbm.at[ubuf[b].at[1]], rows2[b], g2[b]).wait()

    def scatter(b):
        pltpu.async_copy(rows1[b], acc.at[ubuf[b].at[1]], s1[b], add=True)
        pltpu.async_copy(rows2[b], acc.at[ubuf[b].at[0]], s2[b], add=True)

    def scatter_wait(b):
        pltpu.make_async_copy(rows1[b], acc.at[ubuf[b].at[1]], s1[b]).wait()
        pltpu.make_async_copy(rows2[b], acc.at[ubuf[b].at[0]], s2[b]).wait()

    def visit(k, b, pb_unpack, pb_wait=None, pb_pre=None,
              first=False, last=False):
        gather_wait(b)
        scatter(b)
        if not first:
            scatter_wait(1 - b)
        if not last:
            if pb_wait is not None:
                pwait(pb_wait)
            unpack(1 - b, pb_unpack)
            gather(1 - b)
        if pb_pre is not None:
            prefetch(k, pb_pre)

    pltpu.sync_copy(edges_hbm.at[pl.ds(cbase * CHUNK, 2 * CHUNK)], pbuf[0])
    unpack(0, 0)
    gather(0)
    plsc.subcore_barrier()
    visit(0, 0, pb_unpack=0, pb_pre=1, first=True)
    visit(1, 1, pb_unpack=1, pb_wait=1)

    def outer(j, carry):
        k = 2 + 4 * j
        visit(k, 0, pb_unpack=1, pb_pre=0)
        visit(k + 1, 1, pb_unpack=0, pb_wait=0)
        visit(k + 2, 0, pb_unpack=0, pb_pre=1)
        visit(k + 3, 1, pb_unpack=1, pb_wait=1)
        return carry

    lax.fori_loop(0, (nch - 4) // 4, outer, 0)
    visit(0, 0, pb_unpack=1)
    visit(0, 1, pb_unpack=0, last=True)
    scatter_wait(1)

    plsc.subcore_barrier()
    pltpu.sync_copy(acc.at[pl.ds(sid * ZR, ZR)],
                    out_hbm.at[cid, pl.ds(sid * ZR, ZR)])


def kernel(x_n, edge_index, abs_level, rel_level, emb, Wp1, bp1, Wp2, bp2,
           Wm, bm, Wt, bt, Ws, bs, Wo1, bo1, Wo2, bo2):
    del rel_level
    f32 = jnp.float32
    xc = x_n.reshape(N, 1).astype(jnp.int32)
    al = abs_level.reshape(N, 1).astype(jnp.int32)
    pet = _pe_table()

    src = edge_index[0].astype(jnp.int32)
    dst = edge_index[1].astype(jnp.int32)
    pad = jnp.full((EP - E,), N, dtype=jnp.int32)
    srcp = jnp.concatenate([src, pad])
    dstp = jnp.concatenate([dst, pad])
    edges = srcp | (dstp << 16)
    zeros = jnp.zeros((ZR, H), dtype=f32)

    b1 = bp1.reshape(1, H)
    b2 = bp2.reshape(1, H)
    w1e = Wp1[:EMB]
    w1p = Wp1[EMB:]

    node_i32 = pl.BlockSpec((R, 1), lambda i: (i, 0))
    nspec = _node_spec()
    agg_spec = pl.BlockSpec((2, R, H), lambda i: (0, i, 0))
    mshape = jax.ShapeDtypeStruct((NP, H), f32)
    nshape = jax.ShapeDtypeStruct((N, H), f32)

    h0, m1, m2, s = _tc_call(
        _k0_body,
        in_specs=[node_i32, node_i32, _full_spec((NUM_CAT, H)), _full_spec((PE, PE)),
                  _full_spec((EMB, H)), _full_spec((PE, H)), _full_spec((1, H)),
                  _full_spec((H, H)), _full_spec((1, H)),
                  _full_spec((H, H)), _full_spec((1, H)),
                  _full_spec((H, H)), _full_spec((1, H)),
                  _full_spec((H, H)), _full_spec((1, H))],
        out_shapes=(nshape, mshape, mshape, nshape),
        out_specs=(nspec, nspec, nspec, nspec),
    )(xc, al, emb, pet, w1e, w1p, b1, Wp2, b2,
      Wm[0], bm[0].reshape(1, H), Wt[0], bt[0].reshape(1, H),
      Ws[0], bs[0].reshape(1, H))

    hs = [h0]
    for i in range(L - 1):
        agg = _edge_agg(m1, m2, edges, zeros)
        h, m1, m2, s = _tc_call(
            _kmid_body,
            in_specs=[agg_spec, nspec,
                      _full_spec((H, H)), _full_spec((1, H)),
                      _full_spec((H, H)), _full_spec((1, H)),
                      _full_spec((H, H)), _full_spec((1, H))],
            out_shapes=(nshape, mshape, mshape, nshape),
            out_specs=(nspec, nspec, nspec, nspec),
        )(agg, s, Wm[i + 1], bm[i + 1].reshape(1, H),
          Wt[i + 1], bt[i + 1].reshape(1, H),
          Ws[i + 1], bs[i + 1].reshape(1, H))
        hs.append(h)

    agg = _edge_agg(m1, m2, edges, zeros)
    out = _tc_call(
        _kfin_body,
        in_specs=[nspec, nspec, nspec, nspec, agg_spec, nspec,
                  _full_spec(((L + 1) * H, H)), _full_spec((1, H)),
                  _full_spec((H, H)), _full_spec((1, H))],
        out_shapes=nshape,
        out_specs=nspec,
    )(hs[0], hs[1], hs[2], hs[3], agg, s,
      Wo1, bo1.reshape(1, H), Wo2, bo2.reshape(1, H))
    return out
